# baseline (speedup 1.0000x reference)
import jax
import jax.numpy as jnp
from jax.experimental import pallas as pl
from jax.experimental.pallas import tpu as pltpu

N, M, D = 8192, 1024, 1024
BN = 1024


def _body(h_ref, u_ref, w_ref, o_ref):
    hb = h_ref[...]
    ub = u_ref[...]
    w1 = w_ref[0:1, 0:D]
    w2 = w_ref[0:1, D:2 * D]
    w3 = w_ref[0:1, 2 * D:3 * D]
    lhs = hb * w3 + w2
    s = jax.lax.dot_general(
        lhs.astype(jnp.bfloat16), ub,
        dimension_numbers=(((1,), (1,)), ((), ())),
        preferred_element_type=jnp.float32,
    )
    row = jnp.sum(hb * w1, axis=1, keepdims=True)
    o_ref[...] = s + row


def kernel(h, u, w):
    h0, u0 = h[0], u[0].astype(jnp.bfloat16)
    return pl.pallas_call(
        _body,
        grid=(N // BN,),
        in_specs=[
            pl.BlockSpec((BN, D), lambda i: (i, 0)),
            pl.BlockSpec((M, D), lambda i: (0, 0)),
            pl.BlockSpec((1, 3 * D), lambda i: (0, 0)),
        ],
        out_specs=pl.BlockSpec((BN, M), lambda i: (i, 0)),
        out_shape=jax.ShapeDtypeStruct((N, M), jnp.float32),
        compiler_params=pltpu.CompilerParams(
            dimension_semantics=("arbitrary",),
        ),
    )(h0, u0, w)


# speedup vs baseline: 1.0989x; 1.0989x over previous
import jax
import jax.numpy as jnp
from jax.experimental import pallas as pl
from jax.experimental.pallas import tpu as pltpu

N, M, D = 8192, 1024, 1024
BN = 1024


def _body(h_ref, u_ref, w_ref, o_ref, ub_ref):
    @pl.when(pl.program_id(0) == 0)
    def _cast_u():
        ub_ref[...] = u_ref[...].astype(jnp.bfloat16)

    hb = h_ref[...]
    w1 = w_ref[0:1, 0:D]
    w2 = w_ref[0:1, D:2 * D]
    w3 = w_ref[0:1, 2 * D:3 * D]
    lhs = hb * w3 + w2
    s = jax.lax.dot_general(
        lhs.astype(jnp.bfloat16), ub_ref[...],
        dimension_numbers=(((1,), (1,)), ((), ())),
        preferred_element_type=jnp.float32,
    )
    row = jnp.sum(hb * w1, axis=1, keepdims=True)
    o_ref[...] = s + row


def kernel(h, u, w):
    h0, u0 = h[0], u[0]
    return pl.pallas_call(
        _body,
        grid=(N // BN,),
        in_specs=[
            pl.BlockSpec((BN, D), lambda i: (i, 0)),
            pl.BlockSpec((M, D), lambda i: (0, 0)),
            pl.BlockSpec((1, 3 * D), lambda i: (0, 0)),
        ],
        out_specs=pl.BlockSpec((BN, M), lambda i: (i, 0)),
        out_shape=jax.ShapeDtypeStruct((N, M), jnp.float32),
        scratch_shapes=[pltpu.VMEM((M, D), jnp.bfloat16)],
        compiler_params=pltpu.CompilerParams(
            dimension_semantics=("arbitrary",),
        ),
    )(h0, u0, w)


# speedup vs baseline: 1.1122x; 1.0121x over previous
import jax
import jax.numpy as jnp
from jax.experimental import pallas as pl
from jax.experimental.pallas import tpu as pltpu

N, M, D = 8192, 1024, 1024
BN = 1024


def _body(h_ref, u_ref, w_ref, o_ref):
    hb = h_ref[...]
    ub = u_ref[...]
    w1 = w_ref[0:1, 0:D]
    w2 = w_ref[0:1, D:2 * D]
    w3 = w_ref[0:1, 2 * D:3 * D]
    lhs = (hb * w3 + w2).astype(jnp.bfloat16)
    ubt = ub.astype(jnp.bfloat16)
    row = jnp.sum(hb * w1, axis=1, keepdims=True)
    H = BN // 2
    dn = (((1,), (1,)), ((), ()))
    s0 = jax.lax.dot_general(lhs[:H], ubt, dimension_numbers=dn,
                             preferred_element_type=jnp.float32)
    o_ref[0:H, :] = s0 + row[0:H]
    s1 = jax.lax.dot_general(lhs[H:], ubt, dimension_numbers=dn,
                             preferred_element_type=jnp.float32)
    o_ref[H:BN, :] = s1 + row[H:BN]


def kernel(h, u, w):
    h0, u0 = h[0], u[0]
    return pl.pallas_call(
        _body,
        grid=(N // BN,),
        in_specs=[
            pl.BlockSpec((BN, D), lambda i: (i, 0)),
            pl.BlockSpec((M, D), lambda i: (0, 0)),
            pl.BlockSpec((1, 3 * D), lambda i: (0, 0)),
        ],
        out_specs=pl.BlockSpec((BN, M), lambda i: (i, 0)),
        out_shape=jax.ShapeDtypeStruct((N, M), jnp.float32),
        compiler_params=pltpu.CompilerParams(
            dimension_semantics=("arbitrary",),
        ),
    )(h0, u0, w)
